# Initial kernel scaffold; baseline (speedup 1.0000x reference)
import functools

import jax
import jax.numpy as jnp
from jax import lax
from jax.experimental import pallas as pl
from jax.experimental.pallas import tpu as pltpu
from jax.experimental.pallas import tpu_sc as plsc

N_OBJECTS = 100000
NUM_VERTICES = 12
INPUT_DIM = 64
OUTPUT_DIM = 64
BATCH = 16384
ROW = NUM_VERTICES * INPUT_DIM

NC = 2
NS = 16
NW = NC * NS
B_PER_W = BATCH // NW
W_CHUNK = 64
N_CHUNKS = B_PER_W // W_CHUNK

_mesh = plsc.VectorSubcoreMesh(core_axis_name="c", subcore_axis_name="s")


@functools.partial(
    pl.kernel,
    out_type=(
        jax.ShapeDtypeStruct((BATCH, ROW), jnp.float32),
        jax.ShapeDtypeStruct((BATCH, OUTPUT_DIM), jnp.float32),
    ),
    mesh=_mesh,
    scratch_types=[
        pltpu.VMEM((N_CHUNKS, W_CHUNK), jnp.int32),
        pltpu.VMEM((2, W_CHUNK, ROW), jnp.float32),
        pltpu.VMEM((2, W_CHUNK, OUTPUT_DIM), jnp.float32),
        pltpu.SemaphoreType.DMA,
        pltpu.SemaphoreType.DMA,
    ],
)
def _gather_sc(w_hbm, b_hbm, idx_hbm, w_out, b_out, idx_v, wbuf, bbuf, wsem, bsem):
    wid = lax.axis_index("s") * NC + lax.axis_index("c")
    base = wid * B_PER_W
    pltpu.sync_copy(idx_hbm.at[wid], idx_v)
    wcp = pltpu.async_copy(w_hbm.at[idx_v.at[0]], wbuf.at[0], wsem)
    bcp = pltpu.async_copy(b_hbm.at[idx_v.at[0]], bbuf.at[0], bsem)
    for c in range(N_CHUNKS):
        nxt = (c + 1) % 2
        if c + 1 < N_CHUNKS:
            wcp_n = pltpu.async_copy(w_hbm.at[idx_v.at[c + 1]], wbuf.at[nxt], wsem)
            bcp_n = pltpu.async_copy(b_hbm.at[idx_v.at[c + 1]], bbuf.at[nxt], bsem)
        wcp.wait()
        bcp.wait()
        cur = c % 2
        dst = pl.ds(base + c * W_CHUNK, W_CHUNK)
        pltpu.sync_copy(wbuf.at[cur], w_out.at[dst])
        pltpu.sync_copy(bbuf.at[cur], b_out.at[dst])
        if c + 1 < N_CHUNKS:
            wcp, bcp = wcp_n, bcp_n


def kernel(weight, bias, obj_ids):
    w2 = weight.reshape(N_OBJECTS, ROW)
    ids = obj_ids.astype(jnp.int32).reshape(NW, N_CHUNKS, W_CHUNK)
    w_sel, b_sel = _gather_sc(w2, bias, ids)
    return w_sel.reshape(BATCH, NUM_VERTICES, INPUT_DIM), b_sel


# speedup vs baseline: 1.1224x; 1.1224x over previous
import functools

import jax
import jax.numpy as jnp
from jax import lax
from jax.experimental import pallas as pl
from jax.experimental.pallas import tpu as pltpu
from jax.experimental.pallas import tpu_sc as plsc

N_OBJECTS = 100000
NUM_VERTICES = 12
INPUT_DIM = 64
OUTPUT_DIM = 64
BATCH = 16384
ROW = NUM_VERTICES * INPUT_DIM

NC = 2
NS = 16
NW = NC * NS
B_PER_W = BATCH // NW
W_CHUNK = 64
N_CHUNKS = B_PER_W // W_CHUNK

_mesh = plsc.VectorSubcoreMesh(core_axis_name="c", subcore_axis_name="s")


@functools.partial(
    pl.kernel,
    out_type=(
        jax.ShapeDtypeStruct((BATCH, ROW), jnp.float32),
        jax.ShapeDtypeStruct((BATCH, 2 * OUTPUT_DIM), jnp.float32),
    ),
    mesh=_mesh,
    scratch_types=[
        pltpu.VMEM((N_CHUNKS, W_CHUNK), jnp.int32),
        pltpu.VMEM((N_CHUNKS, W_CHUNK), jnp.int32),
        pltpu.VMEM((2, W_CHUNK, ROW), jnp.float32),
        pltpu.VMEM((2, W_CHUNK, 2 * OUTPUT_DIM), jnp.float32),
        pltpu.SemaphoreType.DMA,
        pltpu.SemaphoreType.DMA,
    ],
)
def _gather_sc(w_hbm, b2_hbm, idx_hbm, idx2_hbm, w_out, b_pad_out,
               idx_v, idx2_v, wbuf, bbuf, wsem, bsem):
    wid = lax.axis_index("s") * NC + lax.axis_index("c")
    base = wid * B_PER_W
    pltpu.sync_copy(idx_hbm.at[wid], idx_v)
    pltpu.sync_copy(idx2_hbm.at[wid], idx2_v)
    wcp = pltpu.async_copy(w_hbm.at[idx_v.at[0]], wbuf.at[0], wsem)
    bcp = pltpu.async_copy(b2_hbm.at[idx2_v.at[0]], bbuf.at[0], bsem)
    for c in range(N_CHUNKS):
        nxt = (c + 1) % 2
        if c + 1 < N_CHUNKS:
            wcp_n = pltpu.async_copy(w_hbm.at[idx_v.at[c + 1]], wbuf.at[nxt], wsem)
            bcp_n = pltpu.async_copy(b2_hbm.at[idx2_v.at[c + 1]], bbuf.at[nxt], bsem)
        wcp.wait()
        bcp.wait()
        cur = c % 2
        dst = pl.ds(base + c * W_CHUNK, W_CHUNK)
        pltpu.sync_copy(wbuf.at[cur], w_out.at[dst])
        pltpu.sync_copy(bbuf.at[cur], b_pad_out.at[dst])
        if c + 1 < N_CHUNKS:
            wcp, bcp = wcp_n, bcp_n


_NARROW_BLK = 512


def _narrow_body(pad_ref, par_ref, out_ref):
    x = pad_ref[...]
    p = par_ref[...]
    out_ref[...] = jnp.where(p > 0, x[:, OUTPUT_DIM:], x[:, :OUTPUT_DIM])


def _narrow(b_pad, par):
    return pl.pallas_call(
        _narrow_body,
        grid=(BATCH // _NARROW_BLK,),
        in_specs=[
            pl.BlockSpec((_NARROW_BLK, 2 * OUTPUT_DIM), lambda i: (i, 0)),
            pl.BlockSpec((_NARROW_BLK, 1), lambda i: (i, 0)),
        ],
        out_specs=pl.BlockSpec((_NARROW_BLK, OUTPUT_DIM), lambda i: (i, 0)),
        out_shape=jax.ShapeDtypeStruct((BATCH, OUTPUT_DIM), jnp.float32),
    )(b_pad, par)


def kernel(weight, bias, obj_ids):
    w2 = weight.reshape(N_OBJECTS, ROW)
    b2 = bias.reshape(N_OBJECTS // 2, 2 * OUTPUT_DIM)
    ids = obj_ids.astype(jnp.int32)
    idx = ids.reshape(NW, N_CHUNKS, W_CHUNK)
    idx2 = (ids // 2).reshape(NW, N_CHUNKS, W_CHUNK)
    par = (ids % 2).reshape(BATCH, 1)
    w_sel, b_pad = _gather_sc(w2, b2, idx, idx2)
    b_sel = _narrow(b_pad, par)
    return w_sel.reshape(BATCH, NUM_VERTICES, INPUT_DIM), b_sel
